# baseline (speedup 1.0000x reference)
import functools

import jax
import jax.numpy as jnp
from jax import lax
from jax.experimental import pallas as pl
from jax.experimental.pallas import tpu as pltpu
from jax.experimental.pallas import tpu_sc as plsc

_N = 10000
_D = 128
_TC_ROWS = 5120
_SC_ROWS = _N - _TC_ROWS
_SC_START = _TC_ROWS * _D
_SC_TOTAL = _SC_ROWS * _D
_CPW = 2
_NW = 32
_CHUNK = _SC_TOTAL // (_NW * _CPW)
_LANES = 16
_UNROLL = 5


def _make_sc_add():
    mesh = plsc.VectorSubcoreMesh(core_axis_name="c", subcore_axis_name="s")

    @functools.partial(
        pl.kernel,
        mesh=mesh,
        out_type=jax.ShapeDtypeStruct((_SC_TOTAL,), jnp.float32),
        scratch_types=[
            pltpu.VMEM((_CHUNK,), jnp.float32),
            pltpu.VMEM((_CHUNK,), jnp.float32),
            pltpu.VMEM((_CHUNK,), jnp.float32),
            pltpu.VMEM((_CHUNK,), jnp.float32),
            pltpu.SemaphoreType.DMA,
            pltpu.SemaphoreType.DMA,
            pltpu.SemaphoreType.DMA,
            pltpu.SemaphoreType.DMA,
            pltpu.SemaphoreType.DMA,
            pltpu.SemaphoreType.DMA,
        ],
    )
    def sc_add(x_hbm, pos_hbm, out_hbm,
               bufx0, bufx1, bufp0, bufp1,
               sx0, sx1, sp0, sp1, so0, so1):
        wid = lax.axis_index("s") * 2 + lax.axis_index("c")
        bufx = (bufx0, bufx1)
        bufp = (bufp0, bufp1)
        sx = (sx0, sx1)
        sp = (sp0, sp1)
        so = (so0, so1)

        def start_in(t):
            b = t % 2
            src = _SC_START + (wid * _CPW + t) * _CHUNK
            hx = pltpu.async_copy(x_hbm.at[pl.ds(src, _CHUNK)], bufx[b], sx[b])
            hp = pltpu.async_copy(pos_hbm.at[pl.ds(src, _CHUNK)], bufp[b], sp[b])
            return hx, hp

        def start_out(t):
            b = t % 2
            dst = (wid * _CPW + t) * _CHUNK
            return pltpu.async_copy(bufx[b], out_hbm.at[pl.ds(dst, _CHUNK)], so[b])

        def compute(t):
            b = t % 2
            xv, pv = bufx[b], bufp[b]

            @plsc.parallel_loop(0, _CHUNK, step=_LANES, unroll=_UNROLL)
            def body(i):
                sl = pl.ds(i, _LANES)
                plsc.addupdate(xv.at[sl], pv[sl])

        in_h = {0: start_in(0)}
        out_h = {}
        for t in range(_CPW):
            if t + 1 < _CPW:
                in_h[t + 1] = start_in(t + 1)
            hx, hp = in_h[t]
            hx.wait()
            hp.wait()
            compute(t)
            out_h[t] = start_out(t)
        for t in range(_CPW):
            out_h[t].wait()

    return sc_add


_sc_add = _make_sc_add()


def _tc_add_body(x_ref, p_ref, o_ref):
    o_ref[...] = x_ref[...] + p_ref[...]


def _tc_add(x, pos):
    blk = 320
    grid = (_TC_ROWS // blk,)
    return pl.pallas_call(
        _tc_add_body,
        grid=grid,
        in_specs=[
            pl.BlockSpec((blk, _D), lambda i: (i, 0)),
            pl.BlockSpec((blk, _D), lambda i: (i, 0)),
        ],
        out_specs=pl.BlockSpec((blk, _D), lambda i: (i, 0)),
        out_shape=jax.ShapeDtypeStruct((_N, _D), x.dtype),
    )(x, pos)


def kernel(x, edge_index, pos_embedding):
    sc_part = _sc_add(x.reshape(-1), pos_embedding.reshape(-1))
    tc_out = _tc_add(x, pos_embedding)
    return lax.dynamic_update_slice(
        tc_out, sc_part.reshape(_SC_ROWS, _D), (_TC_ROWS, 0))


# speedup vs baseline: 1.1264x; 1.1264x over previous
import functools

import jax
import jax.numpy as jnp
from jax import lax
from jax.experimental import pallas as pl
from jax.experimental.pallas import tpu as pltpu
from jax.experimental.pallas import tpu_sc as plsc

_N = 10000
_D = 128
_TOTAL = _N * _D
_NW = 32
_CPW = 2
_CHUNK = _TOTAL // (_NW * _CPW)
_LANES = 16
_UNROLL = 5


def _make_sc_add():
    mesh = plsc.VectorSubcoreMesh(core_axis_name="c", subcore_axis_name="s")

    @functools.partial(
        pl.kernel,
        mesh=mesh,
        out_type=jax.ShapeDtypeStruct((_TOTAL,), jnp.float32),
        scratch_types=[
            pltpu.VMEM((_CHUNK,), jnp.float32),
            pltpu.VMEM((_CHUNK,), jnp.float32),
            pltpu.VMEM((_CHUNK,), jnp.float32),
            pltpu.VMEM((_CHUNK,), jnp.float32),
            pltpu.SemaphoreType.DMA,
            pltpu.SemaphoreType.DMA,
            pltpu.SemaphoreType.DMA,
            pltpu.SemaphoreType.DMA,
            pltpu.SemaphoreType.DMA,
            pltpu.SemaphoreType.DMA,
        ],
    )
    def sc_add(x_hbm, pos_hbm, out_hbm,
               bufx0, bufx1, bufp0, bufp1,
               sx0, sx1, sp0, sp1, so0, so1):
        wid = lax.axis_index("s") * 2 + lax.axis_index("c")
        bufx = (bufx0, bufx1)
        bufp = (bufp0, bufp1)
        sx = (sx0, sx1)
        sp = (sp0, sp1)
        so = (so0, so1)

        def start_in(t):
            base = (wid * _CPW + t) * _CHUNK
            hx = pltpu.async_copy(x_hbm.at[pl.ds(base, _CHUNK)], bufx[t], sx[t])
            hp = pltpu.async_copy(pos_hbm.at[pl.ds(base, _CHUNK)], bufp[t], sp[t])
            return hx, hp

        def start_out(t):
            base = (wid * _CPW + t) * _CHUNK
            return pltpu.async_copy(bufx[t], out_hbm.at[pl.ds(base, _CHUNK)], so[t])

        def compute(t):
            xv, pv = bufx[t], bufp[t]

            @plsc.parallel_loop(0, _CHUNK, step=_LANES, unroll=_UNROLL)
            def body(i):
                sl = pl.ds(i, _LANES)
                plsc.addupdate(xv.at[sl], pv[sl])

        in_h = [start_in(t) for t in range(_CPW)]
        out_h = []
        for t in range(_CPW):
            hx, hp = in_h[t]
            hx.wait()
            hp.wait()
            compute(t)
            out_h.append(start_out(t))
        for h in out_h:
            h.wait()

    return sc_add


_sc_add = _make_sc_add()


def kernel(x, edge_index, pos_embedding):
    n, d = x.shape
    out_flat = _sc_add(x.reshape(-1), pos_embedding.reshape(-1))
    return out_flat.reshape(n, d)
